# baseline (speedup 1.0000x reference)
import functools
import jax
import jax.numpy as jnp
from jax import lax
from jax.experimental import pallas as pl
from jax.experimental.pallas import tpu as pltpu
from jax.experimental.pallas import tpu_sc as plsc

N = 1_000_000
N_THR = 19
N_COLS = 21
CH = 400
NCHUNK = N // CH
NG = CH // 16
NW = 32
NPAIR = (NCHUNK + 2 * NW - 1) // (2 * NW)

_mesh = plsc.VectorSubcoreMesh(core_axis_name="c", subcore_axis_name="s")

_GDN = lax.GatherDimensionNumbers(
    offset_dims=(), collapsed_slice_dims=(0,), start_index_map=(0,)
)


def _take16(vec, j):
    idx = jnp.full((16, 1), j, jnp.int32) if isinstance(j, int) else j
    return lax.gather(vec, idx, _GDN, slice_sizes=(1,),
                      mode=lax.GatherScatterMode.PROMISE_IN_BOUNDS)


@functools.partial(
    pl.kernel,
    mesh=_mesh,
    out_type=jax.ShapeDtypeStruct((N, N_COLS), jnp.int32),
    scratch_types=[
        pltpu.VMEM((CH, N_COLS), jnp.int32),
        pltpu.VMEM((CH, N_COLS), jnp.int32),
        pltpu.VMEM((CH,), jnp.float32),
        pltpu.VMEM((CH,), jnp.float32),
        pltpu.VMEM((32,), jnp.float32),
        pltpu.SemaphoreType.DMA,
        pltpu.SemaphoreType.DMA,
        pltpu.SemaphoreType.DMA,
        pltpu.SemaphoreType.DMA,
    ],
)
def _sc_kernel(f_hbm, t_hbm, out_hbm,
               stage_a, stage_b, fb0, fb1, tvm,
               sem_oa, sem_ob, sem_f0, sem_f1):
    wid = lax.axis_index("s") * 2 + lax.axis_index("c")
    lanes = lax.iota(jnp.int32, 16)
    lanes_p5 = lanes + 5
    one16 = jnp.ones((16,), jnp.int32)
    z16 = jnp.zeros((16,), jnp.int32)

    pltpu.sync_copy(t_hbm, tvm.at[pl.ds(0, N_THR)])
    tv0 = tvm[pl.ds(0, 16)]
    tv1 = tvm[pl.ds(16, 16)]
    tsplat = [
        _take16(tv0 if j < 16 else tv1, j % 16)
        for j in range(N_THR)
    ]

    def compute(stage, fb):
        def grp(g, c):
            f = fb[pl.ds(16 * g, 16)]
            acc = z16
            for j in range(N_THR):
                acc = acc + jnp.where(f > tsplat[j], one16, z16)
            for r in range(16):
                b = _take16(acc, r)
                row = 16 * g + r
                stage[row, pl.ds(0, 16)] = jnp.where(lanes == b, one16, z16)
                stage[row, pl.ds(5, 16)] = jnp.where(lanes_p5 == b, one16, z16)
            return c

        lax.fori_loop(0, NG, grp, 0)

    def fetch(k, fb, sem):
        pltpu.make_async_copy(f_hbm.at[pl.ds(k * CH, CH)], fb, sem).start()

    def fetch_wait(fb, sem):
        pltpu.make_async_copy(f_hbm.at[pl.ds(0, CH)], fb, sem).wait()

    def out_start(stage, k, sem):
        pltpu.make_async_copy(
            stage,
            out_hbm.at[pl.ds(k * CH, CH), :],
            sem,
        ).start()

    def out_wait(stage, sem):
        pltpu.make_async_copy(
            stage,
            out_hbm.at[pl.ds(0, CH), :],
            sem,
        ).wait()

    fetch(wid, fb0, sem_f0)

    def pair(p, c):
        ka = wid + 64 * p
        kb = ka + 32

        @pl.when(kb < NCHUNK)
        def _():
            fetch(kb, fb1, sem_f1)

        @pl.when(ka < NCHUNK)
        def _():
            fetch_wait(fb0, sem_f0)

        @pl.when((p > 0) & (ka < NCHUNK))
        def _():
            out_wait(stage_a, sem_oa)

        @pl.when(ka < NCHUNK)
        def _():
            compute(stage_a, fb0)
            out_start(stage_a, ka, sem_oa)

        @pl.when(kb + 32 < NCHUNK)
        def _():
            fetch(kb + 32, fb0, sem_f0)

        @pl.when(kb < NCHUNK)
        def _():
            fetch_wait(fb1, sem_f1)

        @pl.when((p > 0) & (kb < NCHUNK))
        def _():
            out_wait(stage_b, sem_ob)

        @pl.when(kb < NCHUNK)
        def _():
            compute(stage_b, fb1)
            out_start(stage_b, kb, sem_ob)

        return c

    lax.fori_loop(0, NPAIR, pair, 0)
    out_wait(stage_a, sem_oa)
    out_wait(stage_b, sem_ob)


def kernel(feature, thresholds):
    return _sc_kernel(feature, thresholds)


# speedup vs baseline: 1.0521x; 1.0521x over previous
import functools
import jax
import jax.numpy as jnp
from jax import lax
from jax.experimental import pallas as pl
from jax.experimental.pallas import tpu as pltpu
from jax.experimental.pallas import tpu_sc as plsc

N = 1_000_000
N_THR = 19
N_COLS = 21
CH = 400
NCHUNK = N // CH
NG = CH // 16
NW = 16
NPAIR = (NCHUNK + 2 * NW - 1) // (2 * NW)

_mesh = plsc.VectorSubcoreMesh(core_axis_name="c", subcore_axis_name="s", num_cores=1)

_GDN = lax.GatherDimensionNumbers(
    offset_dims=(), collapsed_slice_dims=(0,), start_index_map=(0,)
)


def _take16(vec, j):
    idx = jnp.full((16, 1), j, jnp.int32) if isinstance(j, int) else j
    return lax.gather(vec, idx, _GDN, slice_sizes=(1,),
                      mode=lax.GatherScatterMode.PROMISE_IN_BOUNDS)


@functools.partial(
    pl.kernel,
    mesh=_mesh,
    out_type=jax.ShapeDtypeStruct((N, N_COLS), jnp.int32),
    scratch_types=[
        pltpu.VMEM((CH, N_COLS), jnp.int32),
        pltpu.VMEM((CH, N_COLS), jnp.int32),
        pltpu.VMEM((CH,), jnp.float32),
        pltpu.VMEM((CH,), jnp.float32),
        pltpu.VMEM((32,), jnp.float32),
        pltpu.SemaphoreType.DMA,
        pltpu.SemaphoreType.DMA,
        pltpu.SemaphoreType.DMA,
        pltpu.SemaphoreType.DMA,
    ],
)
def _sc_kernel(f_hbm, t_hbm, out_hbm,
               stage_a, stage_b, fb0, fb1, tvm,
               sem_oa, sem_ob, sem_f0, sem_f1):
    wid = lax.axis_index("s")
    lanes = lax.iota(jnp.int32, 16)
    lanes_p5 = lanes + 5
    one16 = jnp.ones((16,), jnp.int32)
    z16 = jnp.zeros((16,), jnp.int32)

    pltpu.sync_copy(t_hbm, tvm.at[pl.ds(0, N_THR)])
    tv0 = tvm[pl.ds(0, 16)]
    tv1 = tvm[pl.ds(16, 16)]
    tsplat = [
        _take16(tv0 if j < 16 else tv1, j % 16)
        for j in range(N_THR)
    ]

    def compute(stage, fb):
        def grp(g, c):
            f = fb[pl.ds(16 * g, 16)]
            acc = z16
            for j in range(N_THR):
                acc = acc + jnp.where(f > tsplat[j], one16, z16)
            for r in range(16):
                b = _take16(acc, r)
                row = 16 * g + r
                stage[row, pl.ds(0, 16)] = jnp.where(lanes == b, one16, z16)
                stage[row, pl.ds(5, 16)] = jnp.where(lanes_p5 == b, one16, z16)
            return c

        lax.fori_loop(0, NG, grp, 0)

    def fetch(k, fb, sem):
        pltpu.make_async_copy(f_hbm.at[pl.ds(k * CH, CH)], fb, sem).start()

    def fetch_wait(fb, sem):
        pltpu.make_async_copy(f_hbm.at[pl.ds(0, CH)], fb, sem).wait()

    def out_start(stage, k, sem):
        pltpu.make_async_copy(
            stage,
            out_hbm.at[pl.ds(k * CH, CH), :],
            sem,
        ).start()

    def out_wait(stage, sem):
        pltpu.make_async_copy(
            stage,
            out_hbm.at[pl.ds(0, CH), :],
            sem,
        ).wait()

    fetch(wid, fb0, sem_f0)

    def pair(p, c):
        ka = wid + 64 * p
        kb = ka + 32

        @pl.when(kb < NCHUNK)
        def _():
            fetch(kb, fb1, sem_f1)

        @pl.when(ka < NCHUNK)
        def _():
            fetch_wait(fb0, sem_f0)

        @pl.when((p > 0) & (ka < NCHUNK))
        def _():
            out_wait(stage_a, sem_oa)

        @pl.when(ka < NCHUNK)
        def _():
            compute(stage_a, fb0)
            out_start(stage_a, ka, sem_oa)

        @pl.when(kb + 32 < NCHUNK)
        def _():
            fetch(kb + 32, fb0, sem_f0)

        @pl.when(kb < NCHUNK)
        def _():
            fetch_wait(fb1, sem_f1)

        @pl.when((p > 0) & (kb < NCHUNK))
        def _():
            out_wait(stage_b, sem_ob)

        @pl.when(kb < NCHUNK)
        def _():
            compute(stage_b, fb1)
            out_start(stage_b, kb, sem_ob)

        return c

    lax.fori_loop(0, NPAIR, pair, 0)
    out_wait(stage_a, sem_oa)
    out_wait(stage_b, sem_ob)


def kernel(feature, thresholds):
    return _sc_kernel(feature, thresholds)
